# Initial kernel scaffold; baseline (speedup 1.0000x reference)
import jax
import jax.numpy as jnp
from jax.experimental import pallas as pl


def kernel(input_ids, scores, emb, w):
    raise NotImplementedError("write your pallas kernel here")


# speedup vs baseline: 1.4211x; 1.4211x over previous
import jax
import jax.numpy as jnp
from jax import lax
from jax.experimental import pallas as pl
from jax.experimental.pallas import tpu as pltpu
from jax.experimental.pallas import tpu_sc as plsc

_TOPK = 20
_BETA = 1.0
_L = 16
_CH = 64
_CAND_CAP = 2048
_IMIN = -(2 ** 31)
_IBIG = 2 ** 30


def _sc_body(V, T, D, NC, NS):
    nvec = V // _L
    nw = D // _L
    nchunk = T // _CH

    def body(ids_hbm, scores_hbm, emb_hbm, w_hbm, out_hbm,
             ids_v, scores_v, rows_v, cand_rows_v, hist_v,
             cand_key_v, cand_idx_v, topk_idx_v, values_v, w_v,
             sem_s, sem_g):
        b = lax.axis_index("s") * NC + lax.axis_index("c")
        lane = lax.iota(jnp.int32, _L)
        ones_i = jnp.ones((_L,), jnp.int32)
        lane0 = lane == 0

        pltpu.sync_copy(ids_hbm.at[b], ids_v)
        sc_cp = pltpu.async_copy(scores_hbm.at[b], scores_v, sem_s)
        pltpu.sync_copy(w_hbm, w_v)

        def chunk_body(c, acc):
            pltpu.async_copy(
                emb_hbm.at[ids_v.at[pl.ds(c * _CH, _CH)]], rows_v, sem_g
            ).wait()

            def row_body(r, a):
                return tuple(a[j] + rows_v[r, pl.ds(j * _L, _L)]
                             for j in range(nw))
            return lax.fori_loop(0, _CH, row_body, acc)

        acc0 = tuple(jnp.zeros((_L,), jnp.float32) for _ in range(nw))
        acc = lax.fori_loop(0, nchunk, chunk_body, acc0)
        ctx = [a * (1.0 / T) for a in acc]

        sc_cp.wait()

        def sortable(i):
            v = scores_v[pl.ds(i * _L, _L)]
            s = lax.bitcast_convert_type(v, jnp.int32)
            m = jnp.right_shift(s, 31)
            u = jnp.bitwise_xor(s, jnp.bitwise_or(m, jnp.int32(_IMIN)))
            key16 = (plsc.bitcast(u, jnp.uint32) >> 16).astype(jnp.int32)
            skey = jnp.bitwise_xor(u, jnp.int32(_IMIN))
            return key16, skey

        def zero_hist(i, _):
            hist_v[pl.ds(i * _L, _L)] = jnp.zeros((_L,), jnp.int32)
            return 0
        lax.fori_loop(0, 256, zero_hist, 0)

        def h1(i, _):
            key16, _s = sortable(i)
            bin1 = jnp.right_shift(key16, 8)
            plsc.addupdate_scatter(hist_v, [bin1 * _L + lane], ones_i)
            return 0
        lax.fori_loop(0, nvec, h1, 0)

        def scan_hist(target):
            def sbody(i, carry):
                cum, tbin, c_above, found = carry
                bn = 255 - i
                t = jnp.sum(hist_v[pl.ds(bn * _L, _L)])
                hit = jnp.logical_and(jnp.logical_not(found),
                                      cum + t >= target)
                tbin = jnp.where(hit, bn, tbin)
                c_above = jnp.where(hit, cum, c_above)
                return cum + t, tbin, c_above, jnp.logical_or(found, hit)
            _c, tbin, c_above, _f = lax.fori_loop(
                0, 256, sbody,
                (jnp.int32(0), jnp.int32(0), jnp.int32(0), False))
            return tbin, c_above

        t1, ca1 = scan_hist(jnp.int32(_TOPK))

        lax.fori_loop(0, 256, zero_hist, 0)

        def h2(i, _):
            key16, _s = sortable(i)
            keep = jnp.right_shift(key16, 8) == t1
            addr = (key16 & 255) * _L + lane
            plsc.addupdate_scatter(hist_v, [addr], ones_i, mask=keep)
            return 0
        lax.fori_loop(0, nvec, h2, 0)

        t2, _ca2 = scan_hist(jnp.int32(_TOPK) - ca1)
        thresh16 = t1 * 256 + t2

        def coll(i, off):
            key16, skey = sortable(i)
            keep = key16 >= thresh16
            cnt = jnp.sum(keep.astype(jnp.int32))
            plsc.store_compressed(cand_key_v.at[pl.ds(off, _L)], skey,
                                  mask=keep)
            plsc.store_compressed(cand_idx_v.at[pl.ds(off, _L)],
                                  lane + i * _L, mask=keep)
            return jnp.minimum(off + cnt, _CAND_CAP)
        C = lax.fori_loop(0, nvec, coll, jnp.int32(0))

        cand_key_v[pl.ds(C, _L)] = jnp.full((_L,), _IMIN, jnp.int32)
        cand_idx_v[pl.ds(C, _L)] = jnp.full((_L,), _IBIG, jnp.int32)
        nv = (C + _L - 1) // _L

        p0 = lane * 32 + b
        topk_idx_v[pl.ds(0, _L)] = p0
        topk_idx_v[pl.ds(_L, _L)] = p0 + 512

        def pick(k, _):
            def scan_c(i, carry):
                bk, bi, bpos = carry
                v = cand_key_v[pl.ds(i * _L, _L)]
                ix = cand_idx_v[pl.ds(i * _L, _L)]
                m = jnp.max(v)
                im = jnp.min(jnp.where(v == m, ix, jnp.int32(_IBIG)))
                pos = jnp.min(jnp.where(
                    jnp.logical_and(v == m, ix == im), lane + i * _L, jnp.int32(_IBIG)))
                better = jnp.logical_or(
                    m > bk, jnp.logical_and(m == bk, im < bi))
                return (jnp.where(better, m, bk),
                        jnp.where(better, im, bi),
                        jnp.where(better, pos, bpos))
            bk, bi, bpos = lax.fori_loop(0, nv, scan_c,
                (jnp.int32(_IMIN), jnp.int32(_IBIG), jnp.int32(_IBIG)))
            plsc.store_scatter(topk_idx_v, [jnp.broadcast_to(k, (_L,))],
                               jnp.broadcast_to(bi, (_L,)), mask=lane0)
            plsc.store_scatter(cand_key_v, [jnp.broadcast_to(bpos, (_L,))],
                               jnp.full((_L,), _IMIN, jnp.int32), mask=lane0)
            return 0
        lax.fori_loop(0, _TOPK, pick, 0)

        pltpu.async_copy(emb_hbm.at[topk_idx_v], cand_rows_v, sem_g).wait()
        wj = [w_v[pl.ds(j * _L, _L)] for j in range(nw)]

        def val_body(k, _):
            accv = jnp.zeros((_L,), jnp.float32)
            for j in range(nw):
                x = cand_rows_v[k, pl.ds(j * _L, _L)] + ctx[j]
                e = jnp.exp(x + x)
                th = 1.0 - 2.0 / (e + 1.0)
                accv = accv + th * wj[j]
            vk = jnp.sum(accv)
            plsc.store_scatter(values_v, [jnp.broadcast_to(k, (_L,))],
                               jnp.broadcast_to(vk, (_L,)), mask=lane0)
            return 0
        lax.fori_loop(0, _TOPK, val_body, 0)

        v0 = values_v[pl.ds(0, _L)]
        v1 = values_v[pl.ds(_L, _L)]
        mask4 = lane < (_TOPK - _L)
        tot = jnp.sum(v0) + jnp.sum(jnp.where(mask4, v1, 0.0))
        mean = tot * (1.0 / _TOPK)
        i0 = topk_idx_v[pl.ds(0, _L)]
        i1 = topk_idx_v[pl.ds(_L, _L)]
        plsc.addupdate_scatter(scores_v, [i0], (v0 - mean) * _BETA)
        plsc.addupdate_scatter(scores_v, [i1], (v1 - mean) * _BETA,
                               mask=mask4)

        pltpu.sync_copy(scores_v, out_hbm.at[b])

    return body


def kernel(input_ids, scores, emb, w):
    B, V = scores.shape
    T = input_ids.shape[1]
    D = emb.shape[1]
    NC, NS = 2, 16
    assert B == NC * NS
    mesh = plsc.VectorSubcoreMesh(core_axis_name="c", subcore_axis_name="s",
                                  num_cores=NC, num_subcores=NS)
    scratch = [
        pltpu.VMEM((T,), jnp.int32),
        pltpu.VMEM((V,), jnp.float32),
        pltpu.VMEM((_CH, D), jnp.float32),
        pltpu.VMEM((2 * _L, D), jnp.float32),
        pltpu.VMEM((256 * _L,), jnp.int32),
        pltpu.VMEM((_CAND_CAP + _L,), jnp.int32),
        pltpu.VMEM((_CAND_CAP + _L,), jnp.int32),
        pltpu.VMEM((2 * _L,), jnp.int32),
        pltpu.VMEM((2 * _L,), jnp.float32),
        pltpu.VMEM((D,), jnp.float32),
        pltpu.SemaphoreType.DMA,
        pltpu.SemaphoreType.DMA,
    ]
    run = pl.kernel(_sc_body(V, T, D, NC, NS),
                    out_type=jax.ShapeDtypeStruct((B, V), jnp.float32),
                    mesh=mesh, scratch_types=scratch,
                    compiler_params=pltpu.CompilerParams(
                        needs_layout_passes=False))
    return run(input_ids.astype(jnp.int32), scores, emb, w)


# speedup vs baseline: 2.0210x; 1.4221x over previous
import jax
import jax.numpy as jnp
from jax import lax
from jax.experimental import pallas as pl
from jax.experimental.pallas import tpu as pltpu
from jax.experimental.pallas import tpu_sc as plsc

_TOPK = 20
_BETA = 1.0
_L = 16
_CH = 32
_U = 8
_CAND_CAP = 4080
_FCAP = 496
_IMIN = -(2 ** 31)
_IBIG = 2 ** 30


def _sc_body(V, T, D, NC, NS):
    nvec = V // _L
    nw = D // _L
    npair = T // (2 * _CH)

    def body(ids_hbm, scores_hbm, emb_hbm, w_hbm, out_hbm,
             ids_v, scores_v, rows0_v, rows1_v, cand_rows_v, hist_v,
             cand_key_v, cand_idx_v, fkey_v, fidx_v,
             topk_idx_v, values_v, w_v, sem_s, sem_g0, sem_g1):
        b = lax.axis_index("s") * NC + lax.axis_index("c")
        lane = lax.iota(jnp.int32, _L)
        ones_i = jnp.ones((_L,), jnp.int32)
        lane0 = lane == 0

        pltpu.sync_copy(ids_hbm.at[b], ids_v)
        sc_cp = pltpu.async_copy(scores_hbm.at[b], scores_v, sem_s)
        pltpu.sync_copy(w_hbm, w_v)

        def gcopy(c, buf, sem):
            return pltpu.async_copy(
                emb_hbm.at[ids_v.at[pl.ds(c * _CH, _CH)]], buf, sem)

        def acc_rows(buf, a):
            def row_body(r, a):
                a = tuple(a[j] + buf[2 * r, pl.ds(j * _L, _L)]
                          for j in range(nw))
                return tuple(a[j] + buf[2 * r + 1, pl.ds(j * _L, _L)]
                             for j in range(nw))
            return lax.fori_loop(0, _CH // 2, row_body, a)

        gcopy(0, rows0_v, sem_g0)

        def pair_body(p, a):
            c0 = 2 * p
            cp1 = gcopy(c0 + 1, rows1_v, sem_g1)
            pltpu.make_async_copy(
                emb_hbm.at[ids_v.at[pl.ds(c0 * _CH, _CH)]], rows0_v,
                sem_g0).wait()
            a = acc_rows(rows0_v, a)

            @pl.when(p < npair - 1)
            def _():
                gcopy(c0 + 2, rows0_v, sem_g0)
            cp1.wait()
            return acc_rows(rows1_v, a)

        acc0 = tuple(jnp.zeros((_L,), jnp.float32) for _ in range(nw))
        acc = lax.fori_loop(0, npair, pair_body, acc0)
        ctx = [a * (1.0 / T) for a in acc]

        sc_cp.wait()

        def keybits(i):
            v = scores_v[pl.ds(i * _L, _L)]
            s = lax.bitcast_convert_type(v, jnp.int32)
            m = jnp.right_shift(s, 31)
            u = jnp.bitwise_xor(s, jnp.bitwise_or(m, jnp.int32(_IMIN)))
            return plsc.bitcast(u, jnp.uint32)

        def zero_hist(i, _):
            for t in range(_U):
                hist_v[pl.ds((i * _U + t) * _L, _L)] = jnp.zeros(
                    (_L,), jnp.int32)
            return 0
        lax.fori_loop(0, 256 // _U, zero_hist, 0)

        def h1(i8, _):
            for t in range(_U):
                uu = keybits(i8 * _U + t)
                bin1 = (uu >> 24).astype(jnp.int32)
                plsc.addupdate_scatter(hist_v, [bin1 * _L + lane], ones_i)
            return 0
        lax.fori_loop(0, nvec // _U, h1, 0)
        for i in range((nvec // _U) * _U, nvec):
            uu = keybits(i)
            bin1 = (uu >> 24).astype(jnp.int32)
            plsc.addupdate_scatter(hist_v, [bin1 * _L + lane], ones_i)

        def scan_hist(target):
            def sbody(i, carry):
                cum, tbin, c_above, found = carry
                bn = 255 - i
                tt = jnp.sum(hist_v[pl.ds(bn * _L, _L)])
                hit = jnp.logical_and(jnp.logical_not(found),
                                      cum + tt >= target)
                tbin = jnp.where(hit, bn, tbin)
                c_above = jnp.where(hit, cum, c_above)
                return cum + tt, tbin, c_above, jnp.logical_or(found, hit)
            _c, tbin, c_above, _f = lax.fori_loop(
                0, 256, sbody,
                (jnp.int32(0), jnp.int32(0), jnp.int32(0), False))
            return tbin, c_above

        t1, ca1 = scan_hist(jnp.int32(_TOPK))

        def coll(i4, off):
            for t in range(4):
                i = i4 * 4 + t
                uu = keybits(i)
                skey = plsc.bitcast(uu, jnp.int32) ^ jnp.int32(_IMIN)
                bin1 = (uu >> 24).astype(jnp.int32)
                keep = bin1 >= t1
                plsc.store_compressed(cand_key_v.at[pl.ds(off, _L)], skey,
                                      mask=keep)
                plsc.store_compressed(cand_idx_v.at[pl.ds(off, _L)],
                                      lane + i * _L, mask=keep)
                cnt = plsc.all_reduce_population_count(keep)[0]
                off = jnp.minimum(off + cnt, _CAND_CAP)
            return off
        C = lax.fori_loop(0, nvec // 4, coll, jnp.int32(0))
        for i in range((nvec // 4) * 4, nvec):
            uu = keybits(i)
            skey = plsc.bitcast(uu, jnp.int32) ^ jnp.int32(_IMIN)
            bin1 = (uu >> 24).astype(jnp.int32)
            keep = bin1 >= t1
            plsc.store_compressed(cand_key_v.at[pl.ds(C, _L)], skey,
                                  mask=keep)
            plsc.store_compressed(cand_idx_v.at[pl.ds(C, _L)],
                                  lane + i * _L, mask=keep)
            cnt = plsc.all_reduce_population_count(keep)[0]
            C = jnp.minimum(C + cnt, _CAND_CAP)

        cand_key_v[pl.ds(C, _L)] = jnp.full((_L,), _IMIN, jnp.int32)
        cand_idx_v[pl.ds(C, _L)] = jnp.full((_L,), _IBIG, jnp.int32)
        ncv = (C + _L - 1) // _L

        lax.fori_loop(0, 256 // _U, zero_hist, 0)

        def lh(i, _):
            skey = cand_key_v[pl.ds(i * _L, _L)]
            uu = plsc.bitcast(skey ^ jnp.int32(_IMIN), jnp.uint32)
            bin1 = (uu >> 24).astype(jnp.int32)
            bin2 = ((uu >> 16).astype(jnp.int32)) & 255
            plsc.addupdate_scatter(hist_v, [bin2 * _L + lane], ones_i,
                                   mask=bin1 == t1)
            return 0
        lax.fori_loop(0, ncv, lh, 0)

        t2, _ca2 = scan_hist(jnp.int32(_TOPK) - ca1)
        thresh16 = t1 * 256 + t2

        def fc(i, off):
            skey = cand_key_v[pl.ds(i * _L, _L)]
            idx = cand_idx_v[pl.ds(i * _L, _L)]
            uu = plsc.bitcast(skey ^ jnp.int32(_IMIN), jnp.uint32)
            key16 = (uu >> 16).astype(jnp.int32)
            keep = key16 >= thresh16
            plsc.store_compressed(fkey_v.at[pl.ds(off, _L)], skey, mask=keep)
            plsc.store_compressed(fidx_v.at[pl.ds(off, _L)], idx, mask=keep)
            cnt = plsc.all_reduce_population_count(keep)[0]
            return jnp.minimum(off + cnt, _FCAP)
        C2 = lax.fori_loop(0, ncv, fc, jnp.int32(0))

        fkey_v[pl.ds(C2, _L)] = jnp.full((_L,), _IMIN, jnp.int32)
        fidx_v[pl.ds(C2, _L)] = jnp.full((_L,), _IBIG, jnp.int32)
        nv2 = (C2 + _L - 1) // _L

        p0 = lane * 32 + b
        topk_idx_v[pl.ds(0, _L)] = p0
        topk_idx_v[pl.ds(_L, _L)] = p0 + 512

        def pick(k, _):
            def scan_c(i, carry):
                bk, bi, bpos = carry
                v = fkey_v[pl.ds(i * _L, _L)]
                ix = fidx_v[pl.ds(i * _L, _L)]
                m = jnp.max(v)
                im = jnp.min(jnp.where(v == m, ix, jnp.int32(_IBIG)))
                pos = jnp.min(jnp.where(
                    jnp.logical_and(v == m, ix == im), lane + i * _L,
                    jnp.int32(_IBIG)))
                better = jnp.logical_or(
                    m > bk, jnp.logical_and(m == bk, im < bi))
                return (jnp.where(better, m, bk),
                        jnp.where(better, im, bi),
                        jnp.where(better, pos, bpos))
            bk, bi, bpos = lax.fori_loop(
                0, nv2, scan_c,
                (jnp.int32(_IMIN), jnp.int32(_IBIG), jnp.int32(_IBIG)))
            plsc.store_scatter(topk_idx_v, [jnp.broadcast_to(k, (_L,))],
                               jnp.broadcast_to(bi, (_L,)), mask=lane0)
            plsc.store_scatter(fkey_v, [jnp.broadcast_to(bpos, (_L,))],
                               jnp.full((_L,), _IMIN, jnp.int32), mask=lane0)
            return 0
        lax.fori_loop(0, _TOPK, pick, 0)

        pltpu.async_copy(emb_hbm.at[topk_idx_v], cand_rows_v, sem_g0).wait()
        wj = [w_v[pl.ds(j * _L, _L)] for j in range(nw)]

        def val_body(k, _):
            accv = jnp.zeros((_L,), jnp.float32)
            for j in range(nw):
                x = cand_rows_v[k, pl.ds(j * _L, _L)] + ctx[j]
                e = jnp.exp(x + x)
                th = 1.0 - 2.0 / (e + 1.0)
                accv = accv + th * wj[j]
            vk = jnp.sum(accv)
            plsc.store_scatter(values_v, [jnp.broadcast_to(k, (_L,))],
                               jnp.broadcast_to(vk, (_L,)), mask=lane0)
            return 0
        lax.fori_loop(0, _TOPK, val_body, 0)

        v0 = values_v[pl.ds(0, _L)]
        v1 = values_v[pl.ds(_L, _L)]
        mask4 = lane < (_TOPK - _L)
        tot = jnp.sum(v0) + jnp.sum(jnp.where(mask4, v1, 0.0))
        mean = tot * (1.0 / _TOPK)
        i0 = topk_idx_v[pl.ds(0, _L)]
        i1 = topk_idx_v[pl.ds(_L, _L)]
        plsc.addupdate_scatter(scores_v, [i0], (v0 - mean) * _BETA)
        plsc.addupdate_scatter(scores_v, [i1], (v1 - mean) * _BETA,
                               mask=mask4)

        pltpu.sync_copy(scores_v, out_hbm.at[b])

    return body


def kernel(input_ids, scores, emb, w):
    B, V = scores.shape
    T = input_ids.shape[1]
    D = emb.shape[1]
    NC, NS = 2, 16
    assert B == NC * NS
    mesh = plsc.VectorSubcoreMesh(core_axis_name="c", subcore_axis_name="s",
                                  num_cores=NC, num_subcores=NS)
    scratch = [
        pltpu.VMEM((T,), jnp.int32),
        pltpu.VMEM((V,), jnp.float32),
        pltpu.VMEM((_CH, D), jnp.float32),
        pltpu.VMEM((_CH, D), jnp.float32),
        pltpu.VMEM((2 * _L, D), jnp.float32),
        pltpu.VMEM((256 * _L,), jnp.int32),
        pltpu.VMEM((_CAND_CAP + _L,), jnp.int32),
        pltpu.VMEM((_CAND_CAP + _L,), jnp.int32),
        pltpu.VMEM((_FCAP + _L,), jnp.int32),
        pltpu.VMEM((_FCAP + _L,), jnp.int32),
        pltpu.VMEM((2 * _L,), jnp.int32),
        pltpu.VMEM((2 * _L,), jnp.float32),
        pltpu.VMEM((D,), jnp.float32),
        pltpu.SemaphoreType.DMA,
        pltpu.SemaphoreType.DMA,
        pltpu.SemaphoreType.DMA,
    ]
    run = pl.kernel(_sc_body(V, T, D, NC, NS),
                    out_type=jax.ShapeDtypeStruct((B, V), jnp.float32),
                    mesh=mesh, scratch_types=scratch,
                    compiler_params=pltpu.CompilerParams(
                        needs_layout_passes=False))
    return run(input_ids.astype(jnp.int32), scores, emb, w)


# speedup vs baseline: 2.0235x; 1.0013x over previous
import jax
import jax.numpy as jnp
from jax import lax
from jax.experimental import pallas as pl
from jax.experimental.pallas import tpu as pltpu
from jax.experimental.pallas import tpu_sc as plsc

_TOPK = 20
_BETA = 1.0
_L = 16
_CH = 32
_U = 8
_CAND_CAP = 4080
_FCAP = 496
_IMIN = -(2 ** 31)
_IBIG = 2 ** 30


def _sc_body(V, T, D, NC, NS):
    nvec = V // _L
    nw = D // _L
    npair = T // (2 * _CH)

    def body(ids_hbm, scores_hbm, emb_hbm, w_hbm, out_hbm,
             ids_v, scores_v, rows0_v, rows1_v, cand_rows_v, hist_v,
             cand_key_v, cand_idx_v, fkey_v, fidx_v,
             topk_idx_v, values_v, w_v, sem_s, sem_g0, sem_g1):
        b = lax.axis_index("s") * NC + lax.axis_index("c")
        lane = lax.iota(jnp.int32, _L)
        ones_i = jnp.ones((_L,), jnp.int32)
        lane0 = lane == 0

        pltpu.sync_copy(ids_hbm.at[b], ids_v)
        sc_cp = pltpu.async_copy(scores_hbm.at[b], scores_v, sem_s)
        pltpu.sync_copy(w_hbm, w_v)

        _scopeA = jax.named_scope("phaseA_ctx"); _scopeA.__enter__()
        def gcopy(c, buf, sem):
            return pltpu.async_copy(
                emb_hbm.at[ids_v.at[pl.ds(c * _CH, _CH)]], buf, sem)

        def acc_rows(buf, a):
            def row_body(r, a):
                a = tuple(a[j] + buf[2 * r, pl.ds(j * _L, _L)]
                          for j in range(nw))
                return tuple(a[j] + buf[2 * r + 1, pl.ds(j * _L, _L)]
                             for j in range(nw))
            return lax.fori_loop(0, _CH // 2, row_body, a)

        gcopy(0, rows0_v, sem_g0)

        def pair_body(p, a):
            c0 = 2 * p
            cp1 = gcopy(c0 + 1, rows1_v, sem_g1)
            pltpu.make_async_copy(
                emb_hbm.at[ids_v.at[pl.ds(c0 * _CH, _CH)]], rows0_v,
                sem_g0).wait()
            a = acc_rows(rows0_v, a)

            @pl.when(p < npair - 1)
            def _():
                gcopy(c0 + 2, rows0_v, sem_g0)
            cp1.wait()
            return acc_rows(rows1_v, a)

        acc0 = tuple(jnp.zeros((_L,), jnp.float32) for _ in range(nw))
        acc = lax.fori_loop(0, npair, pair_body, acc0)
        ctx = [a * (1.0 / T) for a in acc]

        _scopeA.__exit__(None, None, None)
        sc_cp.wait()

        def keybits(i):
            v = scores_v[pl.ds(i * _L, _L)]
            s = lax.bitcast_convert_type(v, jnp.int32)
            m = jnp.right_shift(s, 31)
            u = jnp.bitwise_xor(s, jnp.bitwise_or(m, jnp.int32(_IMIN)))
            return plsc.bitcast(u, jnp.uint32)

        def zero_hist(i, _):
            for t in range(_U):
                hist_v[pl.ds((i * _U + t) * _L, _L)] = jnp.zeros(
                    (_L,), jnp.int32)
            return 0
        with jax.named_scope("zero1"):
            lax.fori_loop(0, 256 // _U, zero_hist, 0)

        def h1(i8, _):
            for t in range(_U):
                uu = keybits(i8 * _U + t)
                bin1 = (uu >> 24).astype(jnp.int32)
                plsc.addupdate_scatter(hist_v, [bin1 * _L + lane], ones_i)
            return 0
        with jax.named_scope("hist1"):
            lax.fori_loop(0, nvec // _U, h1, 0)
            for i in range((nvec // _U) * _U, nvec):
                uu = keybits(i)
                bin1 = (uu >> 24).astype(jnp.int32)
                plsc.addupdate_scatter(hist_v, [bin1 * _L + lane], ones_i)

        def scan_hist(target):
            def sbody(i, carry):
                cum, tbin, c_above, found = carry
                bn = 255 - i
                tt = jnp.sum(hist_v[pl.ds(bn * _L, _L)])
                hit = jnp.logical_and(jnp.logical_not(found),
                                      cum + tt >= target)
                tbin = jnp.where(hit, bn, tbin)
                c_above = jnp.where(hit, cum, c_above)
                return cum + tt, tbin, c_above, jnp.logical_or(found, hit)
            _c, tbin, c_above, _f = lax.fori_loop(
                0, 256, sbody,
                (jnp.int32(0), jnp.int32(0), jnp.int32(0), False))
            return tbin, c_above

        with jax.named_scope("scan1"):
            t1, ca1 = scan_hist(jnp.int32(_TOPK))

        def coll(i4, off):
            for t in range(4):
                i = i4 * 4 + t
                uu = keybits(i)
                skey = plsc.bitcast(uu, jnp.int32) ^ jnp.int32(_IMIN)
                bin1 = (uu >> 24).astype(jnp.int32)
                keep = bin1 >= t1
                plsc.store_compressed(cand_key_v.at[pl.ds(off, _L)], skey,
                                      mask=keep)
                plsc.store_compressed(cand_idx_v.at[pl.ds(off, _L)],
                                      lane + i * _L, mask=keep)
                cnt = plsc.all_reduce_population_count(keep)[0]
                off = jnp.minimum(off + cnt, _CAND_CAP)
            return off
        _scopeC = jax.named_scope("collect"); _scopeC.__enter__()
        C = lax.fori_loop(0, nvec // 4, coll, jnp.int32(0))
        for i in range((nvec // 4) * 4, nvec):
            uu = keybits(i)
            skey = plsc.bitcast(uu, jnp.int32) ^ jnp.int32(_IMIN)
            bin1 = (uu >> 24).astype(jnp.int32)
            keep = bin1 >= t1
            plsc.store_compressed(cand_key_v.at[pl.ds(C, _L)], skey,
                                  mask=keep)
            plsc.store_compressed(cand_idx_v.at[pl.ds(C, _L)],
                                  lane + i * _L, mask=keep)
            cnt = plsc.all_reduce_population_count(keep)[0]
            C = jnp.minimum(C + cnt, _CAND_CAP)

        _scopeC.__exit__(None, None, None)
        cand_key_v[pl.ds(C, _L)] = jnp.full((_L,), _IMIN, jnp.int32)
        cand_idx_v[pl.ds(C, _L)] = jnp.full((_L,), _IBIG, jnp.int32)
        ncv = (C + _L - 1) // _L

        _scopeL = jax.named_scope("local_refine"); _scopeL.__enter__()
        lax.fori_loop(0, 256 // _U, zero_hist, 0)

        def lh(i, _):
            skey = cand_key_v[pl.ds(i * _L, _L)]
            uu = plsc.bitcast(skey ^ jnp.int32(_IMIN), jnp.uint32)
            bin1 = (uu >> 24).astype(jnp.int32)
            bin2 = ((uu >> 16).astype(jnp.int32)) & 255
            plsc.addupdate_scatter(hist_v, [bin2 * _L + lane], ones_i,
                                   mask=bin1 == t1)
            return 0
        lax.fori_loop(0, ncv, lh, 0)

        t2, _ca2 = scan_hist(jnp.int32(_TOPK) - ca1)
        thresh16 = t1 * 256 + t2

        def fc(i, off):
            skey = cand_key_v[pl.ds(i * _L, _L)]
            idx = cand_idx_v[pl.ds(i * _L, _L)]
            uu = plsc.bitcast(skey ^ jnp.int32(_IMIN), jnp.uint32)
            key16 = (uu >> 16).astype(jnp.int32)
            keep = key16 >= thresh16
            plsc.store_compressed(fkey_v.at[pl.ds(off, _L)], skey, mask=keep)
            plsc.store_compressed(fidx_v.at[pl.ds(off, _L)], idx, mask=keep)
            cnt = plsc.all_reduce_population_count(keep)[0]
            return jnp.minimum(off + cnt, _FCAP)
        C2 = lax.fori_loop(0, ncv, fc, jnp.int32(0))

        fkey_v[pl.ds(C2, _L)] = jnp.full((_L,), _IMIN, jnp.int32)
        fidx_v[pl.ds(C2, _L)] = jnp.full((_L,), _IBIG, jnp.int32)
        nv2 = (C2 + _L - 1) // _L

        p0 = lane * 32 + b
        topk_idx_v[pl.ds(0, _L)] = p0
        topk_idx_v[pl.ds(_L, _L)] = p0 + 512

        def pick(k, _):
            def scan_c(i, carry):
                bk, bi, bpos = carry
                v = fkey_v[pl.ds(i * _L, _L)]
                ix = fidx_v[pl.ds(i * _L, _L)]
                m = jnp.max(v)
                im = jnp.min(jnp.where(v == m, ix, jnp.int32(_IBIG)))
                pos = jnp.min(jnp.where(
                    jnp.logical_and(v == m, ix == im), lane + i * _L,
                    jnp.int32(_IBIG)))
                better = jnp.logical_or(
                    m > bk, jnp.logical_and(m == bk, im < bi))
                return (jnp.where(better, m, bk),
                        jnp.where(better, im, bi),
                        jnp.where(better, pos, bpos))
            bk, bi, bpos = lax.fori_loop(
                0, nv2, scan_c,
                (jnp.int32(_IMIN), jnp.int32(_IBIG), jnp.int32(_IBIG)))
            plsc.store_scatter(topk_idx_v, [jnp.broadcast_to(k, (_L,))],
                               jnp.broadcast_to(bi, (_L,)), mask=lane0)
            plsc.store_scatter(fkey_v, [jnp.broadcast_to(bpos, (_L,))],
                               jnp.full((_L,), _IMIN, jnp.int32), mask=lane0)
            return 0
        lax.fori_loop(0, _TOPK, pick, 0)
        _scopeL.__exit__(None, None, None)

        _scopeV = jax.named_scope("value_head"); _scopeV.__enter__()
        pltpu.async_copy(emb_hbm.at[topk_idx_v], cand_rows_v, sem_g0).wait()
        wj = [w_v[pl.ds(j * _L, _L)] for j in range(nw)]

        def val_body(k, _):
            accv = jnp.zeros((_L,), jnp.float32)
            for j in range(nw):
                x = cand_rows_v[k, pl.ds(j * _L, _L)] + ctx[j]
                e = jnp.exp(x + x)
                th = 1.0 - 2.0 / (e + 1.0)
                accv = accv + th * wj[j]
            vk = jnp.sum(accv)
            plsc.store_scatter(values_v, [jnp.broadcast_to(k, (_L,))],
                               jnp.broadcast_to(vk, (_L,)), mask=lane0)
            return 0
        lax.fori_loop(0, _TOPK, val_body, 0)

        v0 = values_v[pl.ds(0, _L)]
        v1 = values_v[pl.ds(_L, _L)]
        mask4 = lane < (_TOPK - _L)
        tot = jnp.sum(v0) + jnp.sum(jnp.where(mask4, v1, 0.0))
        mean = tot * (1.0 / _TOPK)
        i0 = topk_idx_v[pl.ds(0, _L)]
        i1 = topk_idx_v[pl.ds(_L, _L)]
        plsc.addupdate_scatter(scores_v, [i0], (v0 - mean) * _BETA)
        plsc.addupdate_scatter(scores_v, [i1], (v1 - mean) * _BETA,
                               mask=mask4)

        _scopeV.__exit__(None, None, None)
        with jax.named_scope("row_out"):
            pltpu.sync_copy(scores_v, out_hbm.at[b])

    return body


def kernel(input_ids, scores, emb, w):
    B, V = scores.shape
    T = input_ids.shape[1]
    D = emb.shape[1]
    NC, NS = 2, 16
    assert B == NC * NS
    mesh = plsc.VectorSubcoreMesh(core_axis_name="c", subcore_axis_name="s",
                                  num_cores=NC, num_subcores=NS)
    scratch = [
        pltpu.VMEM((T,), jnp.int32),
        pltpu.VMEM((V,), jnp.float32),
        pltpu.VMEM((_CH, D), jnp.float32),
        pltpu.VMEM((_CH, D), jnp.float32),
        pltpu.VMEM((2 * _L, D), jnp.float32),
        pltpu.VMEM((256 * _L,), jnp.int32),
        pltpu.VMEM((_CAND_CAP + _L,), jnp.int32),
        pltpu.VMEM((_CAND_CAP + _L,), jnp.int32),
        pltpu.VMEM((_FCAP + _L,), jnp.int32),
        pltpu.VMEM((_FCAP + _L,), jnp.int32),
        pltpu.VMEM((2 * _L,), jnp.int32),
        pltpu.VMEM((2 * _L,), jnp.float32),
        pltpu.VMEM((D,), jnp.float32),
        pltpu.SemaphoreType.DMA,
        pltpu.SemaphoreType.DMA,
        pltpu.SemaphoreType.DMA,
    ]
    run = pl.kernel(_sc_body(V, T, D, NC, NS),
                    out_type=jax.ShapeDtypeStruct((B, V), jnp.float32),
                    mesh=mesh, scratch_types=scratch,
                    compiler_params=pltpu.CompilerParams(
                        needs_layout_passes=False))
    return run(input_ids.astype(jnp.int32), scores, emb, w)


# speedup vs baseline: 4.5428x; 2.2450x over previous
import jax
import jax.numpy as jnp
from jax import lax
from jax.experimental import pallas as pl
from jax.experimental.pallas import tpu as pltpu
from jax.experimental.pallas import tpu_sc as plsc

_TOPK = 20
_BETA = 1.0
_L = 16
_CH = 32
_U = 8
_CAND_CAP = 4080
_FCAP = 496
_IMIN = -(2 ** 31)
_IBIG = 2 ** 30


def _sc_body(V, T, D, NC, NS):
    nvec = V // _L
    nw = D // _L
    npair = T // (2 * _CH)

    def body(ids_hbm, scores_hbm, emb_hbm, w_hbm, out_hbm,
             ids_v, scores_v, rows0_v, rows1_v, cand_rows_v, hist_v,
             cand_key_v, cand_idx_v, fkey_v, fidx_v,
             topk_idx_v, values_v, w_v, sem_s, sem_g0, sem_g1):
        b = lax.axis_index("s") * NC + lax.axis_index("c")
        lane = lax.iota(jnp.int32, _L)
        ones_i = jnp.ones((_L,), jnp.int32)
        lane0 = lane == 0

        pltpu.sync_copy(ids_hbm.at[b], ids_v)
        sc_cp = pltpu.async_copy(scores_hbm.at[b], scores_v, sem_s)
        pltpu.sync_copy(w_hbm, w_v)

        _scopeA = jax.named_scope("phaseA_ctx"); _scopeA.__enter__()
        def gcopy(c, buf, sem):
            return pltpu.async_copy(
                emb_hbm.at[ids_v.at[pl.ds(c * _CH, _CH)]], buf, sem)

        def acc_rows(buf, a):
            def row_body(r, a):
                a = tuple(a[j] + buf[2 * r, pl.ds(j * _L, _L)]
                          for j in range(nw))
                return tuple(a[j] + buf[2 * r + 1, pl.ds(j * _L, _L)]
                             for j in range(nw))
            return plsc.parallel_loop(0, _CH // 2, 1, unroll=2,
                                      carry=a)(row_body)

        gcopy(0, rows0_v, sem_g0)

        def pair_body(p, a):
            c0 = 2 * p
            cp1 = gcopy(c0 + 1, rows1_v, sem_g1)
            pltpu.make_async_copy(
                emb_hbm.at[ids_v.at[pl.ds(c0 * _CH, _CH)]], rows0_v,
                sem_g0).wait()
            a = acc_rows(rows0_v, a)

            @pl.when(p < npair - 1)
            def _():
                gcopy(c0 + 2, rows0_v, sem_g0)
            cp1.wait()
            return acc_rows(rows1_v, a)

        acc0 = tuple(jnp.zeros((_L,), jnp.float32) for _ in range(nw))
        acc = lax.fori_loop(0, npair, pair_body, acc0)
        ctx = [a * (1.0 / T) for a in acc]

        _scopeA.__exit__(None, None, None)
        sc_cp.wait()

        def keybits(i):
            v = scores_v[pl.ds(i * _L, _L)]
            s = lax.bitcast_convert_type(v, jnp.int32)
            m = jnp.right_shift(s, 31)
            u = jnp.bitwise_xor(s, jnp.bitwise_or(m, jnp.int32(_IMIN)))
            return plsc.bitcast(u, jnp.uint32)

        def zero_hist(i):
            hist_v[pl.ds(i * _L, _L)] = jnp.zeros((_L,), jnp.int32)
        with jax.named_scope("zero1"):
            plsc.parallel_loop(0, 256, 1, unroll=8)(zero_hist)

        def h1(i):
            uu = keybits(i)
            bin1 = (uu >> 24).astype(jnp.int32)
            plsc.addupdate_scatter(hist_v, [bin1 * _L + lane], ones_i)
        with jax.named_scope("hist1"):
            plsc.parallel_loop(0, nvec, 1, unroll=_U)(h1)

        def scan_hist(target):
            def sbody(i, carry):
                cum, tbin, c_above, found = carry
                bn = 255 - i
                tt = jnp.sum(hist_v[pl.ds(bn * _L, _L)])
                hit = jnp.logical_and(jnp.logical_not(found),
                                      cum + tt >= target)
                tbin = jnp.where(hit, bn, tbin)
                c_above = jnp.where(hit, cum, c_above)
                return cum + tt, tbin, c_above, jnp.logical_or(found, hit)
            _c, tbin, c_above, _f = plsc.parallel_loop(
                0, 256, 1, unroll=4,
                carry=(jnp.int32(0), jnp.int32(0), jnp.int32(0),
                       jnp.bool_(False)))(sbody)
            return tbin, c_above

        with jax.named_scope("scan1"):
            t1, ca1 = scan_hist(jnp.int32(_TOPK))

        def coll(i, off):
            uu = keybits(i)
            skey = plsc.bitcast(uu, jnp.int32) ^ jnp.int32(_IMIN)
            bin1 = (uu >> 24).astype(jnp.int32)
            keep = bin1 >= t1
            plsc.store_compressed(cand_key_v.at[pl.ds(off, _L)], skey,
                                  mask=keep)
            plsc.store_compressed(cand_idx_v.at[pl.ds(off, _L)],
                                  lane + i * _L, mask=keep)
            cnt = plsc.all_reduce_population_count(keep)[0]
            return jnp.minimum(off + cnt, _CAND_CAP)
        _scopeC = jax.named_scope("collect"); _scopeC.__enter__()
        C = plsc.parallel_loop(0, nvec, 1, unroll=4,
                               carry=jnp.int32(0))(coll)

        _scopeC.__exit__(None, None, None)
        cand_key_v[pl.ds(C, _L)] = jnp.full((_L,), _IMIN, jnp.int32)
        cand_idx_v[pl.ds(C, _L)] = jnp.full((_L,), _IBIG, jnp.int32)
        ncv = (C + _L - 1) // _L

        _scopeL = jax.named_scope("local_refine"); _scopeL.__enter__()
        plsc.parallel_loop(0, 256, 1, unroll=8)(zero_hist)

        def lh(i):
            skey = cand_key_v[pl.ds(i * _L, _L)]
            uu = plsc.bitcast(skey ^ jnp.int32(_IMIN), jnp.uint32)
            bin1 = (uu >> 24).astype(jnp.int32)
            bin2 = ((uu >> 16).astype(jnp.int32)) & 255
            plsc.addupdate_scatter(hist_v, [bin2 * _L + lane], ones_i,
                                   mask=bin1 == t1)
        plsc.parallel_loop(0, ncv, 1, unroll=4)(lh)

        t2, _ca2 = scan_hist(jnp.int32(_TOPK) - ca1)
        thresh16 = t1 * 256 + t2

        def fc(i, off):
            skey = cand_key_v[pl.ds(i * _L, _L)]
            idx = cand_idx_v[pl.ds(i * _L, _L)]
            uu = plsc.bitcast(skey ^ jnp.int32(_IMIN), jnp.uint32)
            key16 = (uu >> 16).astype(jnp.int32)
            keep = key16 >= thresh16
            plsc.store_compressed(fkey_v.at[pl.ds(off, _L)], skey, mask=keep)
            plsc.store_compressed(fidx_v.at[pl.ds(off, _L)], idx, mask=keep)
            cnt = plsc.all_reduce_population_count(keep)[0]
            return jnp.minimum(off + cnt, _FCAP)
        C2 = plsc.parallel_loop(0, ncv, 1, unroll=4,
                                carry=jnp.int32(0))(fc)

        fkey_v[pl.ds(C2, _L)] = jnp.full((_L,), _IMIN, jnp.int32)
        fidx_v[pl.ds(C2, _L)] = jnp.full((_L,), _IBIG, jnp.int32)
        nv2 = (C2 + _L - 1) // _L

        p0 = lane * 32 + b
        topk_idx_v[pl.ds(0, _L)] = p0
        topk_idx_v[pl.ds(_L, _L)] = p0 + 512

        def pick(k, _):
            def scan_c(i, carry):
                bk, bi, bpos = carry
                v = fkey_v[pl.ds(i * _L, _L)]
                ix = fidx_v[pl.ds(i * _L, _L)]
                m = jnp.max(v)
                im = jnp.min(jnp.where(v == m, ix, jnp.int32(_IBIG)))
                pos = jnp.min(jnp.where(
                    jnp.logical_and(v == m, ix == im), lane + i * _L,
                    jnp.int32(_IBIG)))
                better = jnp.logical_or(
                    m > bk, jnp.logical_and(m == bk, im < bi))
                return (jnp.where(better, m, bk),
                        jnp.where(better, im, bi),
                        jnp.where(better, pos, bpos))
            bk, bi, bpos = plsc.parallel_loop(
                0, nv2, 1, unroll=2,
                carry=(jnp.int32(_IMIN), jnp.int32(_IBIG),
                       jnp.int32(_IBIG)))(scan_c)
            plsc.store_scatter(topk_idx_v, [jnp.broadcast_to(k, (_L,))],
                               jnp.broadcast_to(bi, (_L,)), mask=lane0)
            plsc.store_scatter(fkey_v, [jnp.broadcast_to(bpos, (_L,))],
                               jnp.full((_L,), _IMIN, jnp.int32), mask=lane0)
            return 0
        lax.fori_loop(0, _TOPK, pick, 0)
        _scopeL.__exit__(None, None, None)

        _scopeV = jax.named_scope("value_head"); _scopeV.__enter__()
        pltpu.async_copy(emb_hbm.at[topk_idx_v], cand_rows_v, sem_g0).wait()
        wj = [w_v[pl.ds(j * _L, _L)] for j in range(nw)]

        def val_body(k, _):
            accv = jnp.zeros((_L,), jnp.float32)
            for j in range(nw):
                x = cand_rows_v[k, pl.ds(j * _L, _L)] + ctx[j]
                e = jnp.exp(x + x)
                th = 1.0 - 2.0 / (e + 1.0)
                accv = accv + th * wj[j]
            vk = jnp.sum(accv)
            plsc.store_scatter(values_v, [jnp.broadcast_to(k, (_L,))],
                               jnp.broadcast_to(vk, (_L,)), mask=lane0)
            return 0
        lax.fori_loop(0, _TOPK, val_body, 0)

        v0 = values_v[pl.ds(0, _L)]
        v1 = values_v[pl.ds(_L, _L)]
        mask4 = lane < (_TOPK - _L)
        tot = jnp.sum(v0) + jnp.sum(jnp.where(mask4, v1, 0.0))
        mean = tot * (1.0 / _TOPK)
        i0 = topk_idx_v[pl.ds(0, _L)]
        i1 = topk_idx_v[pl.ds(_L, _L)]
        plsc.addupdate_scatter(scores_v, [i0], (v0 - mean) * _BETA)
        plsc.addupdate_scatter(scores_v, [i1], (v1 - mean) * _BETA,
                               mask=mask4)

        _scopeV.__exit__(None, None, None)
        with jax.named_scope("row_out"):
            pltpu.sync_copy(scores_v, out_hbm.at[b])

    return body


def kernel(input_ids, scores, emb, w):
    B, V = scores.shape
    T = input_ids.shape[1]
    D = emb.shape[1]
    NC, NS = 2, 16
    assert B == NC * NS
    mesh = plsc.VectorSubcoreMesh(core_axis_name="c", subcore_axis_name="s",
                                  num_cores=NC, num_subcores=NS)
    scratch = [
        pltpu.VMEM((T,), jnp.int32),
        pltpu.VMEM((V,), jnp.float32),
        pltpu.VMEM((_CH, D), jnp.float32),
        pltpu.VMEM((_CH, D), jnp.float32),
        pltpu.VMEM((2 * _L, D), jnp.float32),
        pltpu.VMEM((256 * _L,), jnp.int32),
        pltpu.VMEM((_CAND_CAP + _L,), jnp.int32),
        pltpu.VMEM((_CAND_CAP + _L,), jnp.int32),
        pltpu.VMEM((_FCAP + _L,), jnp.int32),
        pltpu.VMEM((_FCAP + _L,), jnp.int32),
        pltpu.VMEM((2 * _L,), jnp.int32),
        pltpu.VMEM((2 * _L,), jnp.float32),
        pltpu.VMEM((D,), jnp.float32),
        pltpu.SemaphoreType.DMA,
        pltpu.SemaphoreType.DMA,
        pltpu.SemaphoreType.DMA,
    ]
    run = pl.kernel(_sc_body(V, T, D, NC, NS),
                    out_type=jax.ShapeDtypeStruct((B, V), jnp.float32),
                    mesh=mesh, scratch_types=scratch,
                    compiler_params=pltpu.CompilerParams(
                        needs_layout_passes=False))
    return run(input_ids.astype(jnp.int32), scores, emb, w)


# speedup vs baseline: 5.4319x; 1.1957x over previous
import jax
import jax.numpy as jnp
from jax import lax
from jax.experimental import pallas as pl
from jax.experimental.pallas import tpu as pltpu
from jax.experimental.pallas import tpu_sc as plsc

_TOPK = 20
_BETA = 1.0
_L = 16
_CH = 16
_NB = 4
_NQ1 = 4
_CAND_CAP = 3056
_FCAP = 496
_IMIN = -(2 ** 31)
_IBIG = 2 ** 30


def _sc_body(V, T, D, NC, NS):
    nvec = V // _L
    nw = D // _L
    nchunk = T // _CH
    nquad = nchunk // _NB
    nq_h = (nquad - _NQ1) // 2
    q_hist0, q_coll0 = _NQ1, _NQ1 + nq_h
    sl_h = -(-nvec // nq_h)
    sl_c = -(-nvec // (nquad - q_coll0))

    def body(ids_hbm, scores_hbm, emb_hbm, w_hbm, out_hbm,
             ids_v, scores_v, rows0_v, rows1_v, rows2_v, rows3_v,
             cand_rows_v, hist_v, cand_key_v, cand_idx_v, fkey_v, fidx_v,
             topk_idx_v, values_v, w_v,
             sem_s, sem_g0, sem_g1, sem_g2, sem_g3):
        b = lax.axis_index("s") * NC + lax.axis_index("c")
        lane = lax.iota(jnp.int32, _L)
        ones_i = jnp.ones((_L,), jnp.int32)
        lane0 = lane == 0

        pltpu.sync_copy(ids_hbm.at[b], ids_v)
        sc_cp = pltpu.async_copy(scores_hbm.at[b], scores_v, sem_s)
        pltpu.sync_copy(w_hbm, w_v)

        def gcopy(c, buf, sem):
            return pltpu.async_copy(
                emb_hbm.at[ids_v.at[pl.ds(c * _CH, _CH)]], buf, sem)

        def acc_rows(buf, a):
            def row_body(r, a):
                a = tuple(a[j] + buf[2 * r, pl.ds(j * _L, _L)]
                          for j in range(nw))
                return tuple(a[j] + buf[2 * r + 1, pl.ds(j * _L, _L)]
                             for j in range(nw))
            return plsc.parallel_loop(0, _CH // 2, 1, unroll=2,
                                      carry=a)(row_body)

        bufs = (rows0_v, rows1_v, rows2_v, rows3_v)
        sems = (sem_g0, sem_g1, sem_g2, sem_g3)
        for q in range(_NB):
            gcopy(q, bufs[q], sems[q])

        def quad_step(p, a):
            for q in range(_NB):
                c = p * _NB + q
                pltpu.make_async_copy(
                    emb_hbm.at[ids_v.at[pl.ds(c * _CH, _CH)]], bufs[q],
                    sems[q]).wait()
                a = acc_rows(bufs[q], a)

                @pl.when(c + _NB < nchunk)
                def _():
                    gcopy(c + _NB, bufs[q], sems[q])
            return a

        def keybits(i):
            v = scores_v[pl.ds(i * _L, _L)]
            s = lax.bitcast_convert_type(v, jnp.int32)
            m = jnp.right_shift(s, 31)
            u = jnp.bitwise_xor(s, jnp.bitwise_or(m, jnp.int32(_IMIN)))
            return plsc.bitcast(u, jnp.uint32)

        def zero_hist(i):
            hist_v[pl.ds(i * _L, _L)] = jnp.zeros((_L,), jnp.int32)

        def h1(i):
            uu = keybits(i)
            bin1 = (uu >> 24).astype(jnp.int32)
            plsc.addupdate_scatter(hist_v, [bin1 * _L + lane], ones_i)

        def scan_hist(target):
            def sbody(i, carry):
                cum, tbin, c_above, found = carry
                bn = 255 - i
                tt = jnp.sum(hist_v[pl.ds(bn * _L, _L)])
                hit = jnp.logical_and(jnp.logical_not(found),
                                      cum + tt >= target)
                tbin = jnp.where(hit, bn, tbin)
                c_above = jnp.where(hit, cum, c_above)
                return cum + tt, tbin, c_above, jnp.logical_or(found, hit)
            _c, tbin, c_above, _f = plsc.parallel_loop(
                0, 256, 1, unroll=4,
                carry=(jnp.int32(0), jnp.int32(0), jnp.int32(0),
                       jnp.bool_(False)))(sbody)
            return tbin, c_above

        acc0 = tuple(jnp.zeros((_L,), jnp.float32) for _ in range(nw))
        acc = lax.fori_loop(0, _NQ1, quad_step, acc0)

        sc_cp.wait()
        plsc.parallel_loop(0, 256, 1, unroll=8)(zero_hist)

        def quad_hist(p, a):
            a = quad_step(p, a)
            lo = (p - q_hist0) * sl_h
            plsc.parallel_loop(lo, jnp.minimum(lo + sl_h, nvec), 1,
                               unroll=_NB)(h1)
            return a
        acc = lax.fori_loop(q_hist0, q_coll0, quad_hist, acc)

        t1, ca1 = scan_hist(jnp.int32(_TOPK))

        def coll(i, off):
            uu = keybits(i)
            skey = plsc.bitcast(uu, jnp.int32) ^ jnp.int32(_IMIN)
            bin1 = (uu >> 24).astype(jnp.int32)
            keep = bin1 >= t1
            plsc.store_compressed(cand_key_v.at[pl.ds(off, _L)], skey,
                                  mask=keep)
            plsc.store_compressed(cand_idx_v.at[pl.ds(off, _L)],
                                  lane + i * _L, mask=keep)
            cnt = plsc.all_reduce_population_count(keep)[0]
            return jnp.minimum(off + cnt, _CAND_CAP)

        def quad_coll(p, carry):
            a, off = carry
            a = quad_step(p, a)
            lo = (p - q_coll0) * sl_c
            off = plsc.parallel_loop(lo, jnp.minimum(lo + sl_c, nvec), 1,
                                     unroll=4, carry=off)(coll)
            return a, off
        acc, C = lax.fori_loop(q_coll0, nquad, quad_coll,
                               (acc, jnp.int32(0)))
        ctx = [a * (1.0 / T) for a in acc]

        cand_key_v[pl.ds(C, _L)] = jnp.full((_L,), _IMIN, jnp.int32)
        cand_idx_v[pl.ds(C, _L)] = jnp.full((_L,), _IBIG, jnp.int32)
        ncv = (C + _L - 1) // _L

        plsc.parallel_loop(0, 256, 1, unroll=8)(zero_hist)

        def lh(i):
            skey = cand_key_v[pl.ds(i * _L, _L)]
            uu = plsc.bitcast(skey ^ jnp.int32(_IMIN), jnp.uint32)
            bin1 = (uu >> 24).astype(jnp.int32)
            bin2 = ((uu >> 16).astype(jnp.int32)) & 255
            plsc.addupdate_scatter(hist_v, [bin2 * _L + lane], ones_i,
                                   mask=bin1 == t1)
        plsc.parallel_loop(0, ncv, 1, unroll=4)(lh)

        t2, _ca2 = scan_hist(jnp.int32(_TOPK) - ca1)
        thresh16 = t1 * 256 + t2

        def fc(i, off):
            skey = cand_key_v[pl.ds(i * _L, _L)]
            idx = cand_idx_v[pl.ds(i * _L, _L)]
            uu = plsc.bitcast(skey ^ jnp.int32(_IMIN), jnp.uint32)
            key16 = (uu >> 16).astype(jnp.int32)
            keep = key16 >= thresh16
            plsc.store_compressed(fkey_v.at[pl.ds(off, _L)], skey, mask=keep)
            plsc.store_compressed(fidx_v.at[pl.ds(off, _L)], idx, mask=keep)
            cnt = plsc.all_reduce_population_count(keep)[0]
            return jnp.minimum(off + cnt, _FCAP)
        C2 = plsc.parallel_loop(0, ncv, 1, unroll=4,
                                carry=jnp.int32(0))(fc)

        fkey_v[pl.ds(C2, _L)] = jnp.full((_L,), _IMIN, jnp.int32)
        fidx_v[pl.ds(C2, _L)] = jnp.full((_L,), _IBIG, jnp.int32)
        nv2 = (C2 + _L - 1) // _L

        p0 = lane * 32 + b
        topk_idx_v[pl.ds(0, _L)] = p0
        topk_idx_v[pl.ds(_L, _L)] = p0 + 512

        def pick(k, _):
            def scan_c(i, carry):
                bk, bi, bpos = carry
                v = fkey_v[pl.ds(i * _L, _L)]
                ix = fidx_v[pl.ds(i * _L, _L)]
                m = jnp.max(v)
                im = jnp.min(jnp.where(v == m, ix, jnp.int32(_IBIG)))
                pos = jnp.min(jnp.where(
                    jnp.logical_and(v == m, ix == im), lane + i * _L,
                    jnp.int32(_IBIG)))
                better = jnp.logical_or(
                    m > bk, jnp.logical_and(m == bk, im < bi))
                return (jnp.where(better, m, bk),
                        jnp.where(better, im, bi),
                        jnp.where(better, pos, bpos))
            bk, bi, bpos = plsc.parallel_loop(
                0, nv2, 1, unroll=2,
                carry=(jnp.int32(_IMIN), jnp.int32(_IBIG),
                       jnp.int32(_IBIG)))(scan_c)
            plsc.store_scatter(topk_idx_v, [jnp.broadcast_to(k, (_L,))],
                               jnp.broadcast_to(bi, (_L,)), mask=lane0)
            plsc.store_scatter(fkey_v, [jnp.broadcast_to(bpos, (_L,))],
                               jnp.full((_L,), _IMIN, jnp.int32), mask=lane0)
            return 0
        lax.fori_loop(0, _TOPK, pick, 0)

        pltpu.async_copy(emb_hbm.at[topk_idx_v], cand_rows_v, sem_g0).wait()
        wj = [w_v[pl.ds(j * _L, _L)] for j in range(nw)]

        def val_body(k, _):
            accv = jnp.zeros((_L,), jnp.float32)
            for j in range(nw):
                x = cand_rows_v[k, pl.ds(j * _L, _L)] + ctx[j]
                e = jnp.exp(x + x)
                th = 1.0 - 2.0 / (e + 1.0)
                accv = accv + th * wj[j]
            vk = jnp.sum(accv)
            plsc.store_scatter(values_v, [jnp.broadcast_to(k, (_L,))],
                               jnp.broadcast_to(vk, (_L,)), mask=lane0)
            return 0
        lax.fori_loop(0, _TOPK, val_body, 0)

        v0 = values_v[pl.ds(0, _L)]
        v1 = values_v[pl.ds(_L, _L)]
        mask4 = lane < (_TOPK - _L)
        tot = jnp.sum(v0) + jnp.sum(jnp.where(mask4, v1, 0.0))
        mean = tot * (1.0 / _TOPK)
        i0 = topk_idx_v[pl.ds(0, _L)]
        i1 = topk_idx_v[pl.ds(_L, _L)]
        plsc.addupdate_scatter(scores_v, [i0], (v0 - mean) * _BETA)
        plsc.addupdate_scatter(scores_v, [i1], (v1 - mean) * _BETA,
                               mask=mask4)

        pltpu.sync_copy(scores_v, out_hbm.at[b])

    return body


def kernel(input_ids, scores, emb, w):
    B, V = scores.shape
    T = input_ids.shape[1]
    D = emb.shape[1]
    NC, NS = 2, 16
    assert B == NC * NS
    mesh = plsc.VectorSubcoreMesh(core_axis_name="c", subcore_axis_name="s",
                                  num_cores=NC, num_subcores=NS)
    scratch = [
        pltpu.VMEM((T,), jnp.int32),
        pltpu.VMEM((V,), jnp.float32),
        pltpu.VMEM((_CH, D), jnp.float32),
        pltpu.VMEM((_CH, D), jnp.float32),
        pltpu.VMEM((_CH, D), jnp.float32),
        pltpu.VMEM((_CH, D), jnp.float32),
        pltpu.VMEM((2 * _L, D), jnp.float32),
        pltpu.VMEM((256 * _L,), jnp.int32),
        pltpu.VMEM((_CAND_CAP + _L,), jnp.int32),
        pltpu.VMEM((_CAND_CAP + _L,), jnp.int32),
        pltpu.VMEM((_FCAP + _L,), jnp.int32),
        pltpu.VMEM((_FCAP + _L,), jnp.int32),
        pltpu.VMEM((2 * _L,), jnp.int32),
        pltpu.VMEM((2 * _L,), jnp.float32),
        pltpu.VMEM((D,), jnp.float32),
        pltpu.SemaphoreType.DMA,
        pltpu.SemaphoreType.DMA,
        pltpu.SemaphoreType.DMA,
        pltpu.SemaphoreType.DMA,
        pltpu.SemaphoreType.DMA,
    ]
    run = pl.kernel(_sc_body(V, T, D, NC, NS),
                    out_type=jax.ShapeDtypeStruct((B, V), jnp.float32),
                    mesh=mesh, scratch_types=scratch,
                    compiler_params=pltpu.CompilerParams(
                        needs_layout_passes=False))
    return run(input_ids.astype(jnp.int32), scores, emb, w)


# speedup vs baseline: 5.4765x; 1.0082x over previous
import jax
import jax.numpy as jnp
from jax import lax
from jax.experimental import pallas as pl
from jax.experimental.pallas import tpu as pltpu
from jax.experimental.pallas import tpu_sc as plsc

_TOPK = 20
_BETA = 1.0
_L = 16
_CH = 16
_NB = 4
_NQ1 = 4
_CAND_CAP = 3056
_FCAP = 496
_IMIN = -(2 ** 31)
_IBIG = 2 ** 30


def _sc_body(V, T, D, NC, NS):
    nvec = V // _L
    nw = D // _L
    nchunk = T // _CH
    nquad = nchunk // _NB
    nq_h = (nquad - _NQ1) // 2
    q_hist0, q_coll0 = _NQ1, _NQ1 + nq_h
    sl_h = -(-nvec // nq_h)
    sl_c = -(-nvec // (nquad - q_coll0))

    def body(ids_hbm, scores_hbm, emb_hbm, w_hbm, out_hbm,
             ids_v, scores_v, rows0_v, rows1_v, rows2_v, rows3_v,
             cand_rows_v, hist_v, cand_key_v, cand_idx_v, fkey_v, fidx_v,
             topk_idx_v, values_v, w_v,
             sem_s, sem_g0, sem_g1, sem_g2, sem_g3):
        b = lax.axis_index("s") * NC + lax.axis_index("c")
        lane = lax.iota(jnp.int32, _L)
        ones_i = jnp.ones((_L,), jnp.int32)
        lane0 = lane == 0

        ids_cp = pltpu.async_copy(ids_hbm.at[b], ids_v, sem_g0)
        sc_cp = pltpu.async_copy(scores_hbm.at[b], scores_v, sem_s)
        pltpu.sync_copy(w_hbm, w_v)
        ids_cp.wait()

        def gcopy(c, buf, sem):
            return pltpu.async_copy(
                emb_hbm.at[ids_v.at[pl.ds(c * _CH, _CH)]], buf, sem)

        def acc_rows(buf, a):
            def row_body(r, a):
                a = tuple(a[j] + buf[2 * r, pl.ds(j * _L, _L)]
                          for j in range(nw))
                return tuple(a[j] + buf[2 * r + 1, pl.ds(j * _L, _L)]
                             for j in range(nw))
            return plsc.parallel_loop(0, _CH // 2, 1, unroll=2,
                                      carry=a)(row_body)

        bufs = (rows0_v, rows1_v, rows2_v, rows3_v)
        sems = (sem_g0, sem_g1, sem_g2, sem_g3)
        for q in range(_NB):
            gcopy(q, bufs[q], sems[q])

        def quad_step(p, a):
            for q in range(_NB):
                c = p * _NB + q
                pltpu.make_async_copy(
                    emb_hbm.at[ids_v.at[pl.ds(c * _CH, _CH)]], bufs[q],
                    sems[q]).wait()
                a = acc_rows(bufs[q], a)

                @pl.when(c + _NB < nchunk)
                def _():
                    gcopy(c + _NB, bufs[q], sems[q])
            return a

        def keybits(i):
            v = scores_v[pl.ds(i * _L, _L)]
            s = lax.bitcast_convert_type(v, jnp.int32)
            m = jnp.right_shift(s, 31)
            u = jnp.bitwise_xor(s, jnp.bitwise_or(m, jnp.int32(_IMIN)))
            return plsc.bitcast(u, jnp.uint32)

        def zero_hist(i):
            hist_v[pl.ds(i * _L, _L)] = jnp.zeros((_L,), jnp.int32)

        def h1(i):
            uu = keybits(i)
            bin1 = (uu >> 24).astype(jnp.int32)
            plsc.addupdate_scatter(hist_v, [bin1 * _L + lane], ones_i)

        def scan_hist(target):
            def sbody(i, carry):
                cum, tbin, c_above, found = carry
                bn = 255 - i
                tt = jnp.sum(hist_v[pl.ds(bn * _L, _L)])
                hit = jnp.logical_and(jnp.logical_not(found),
                                      cum + tt >= target)
                tbin = jnp.where(hit, bn, tbin)
                c_above = jnp.where(hit, cum, c_above)
                return cum + tt, tbin, c_above, jnp.logical_or(found, hit)
            _c, tbin, c_above, _f = plsc.parallel_loop(
                0, 256, 1, unroll=4,
                carry=(jnp.int32(0), jnp.int32(0), jnp.int32(0),
                       jnp.bool_(False)))(sbody)
            return tbin, c_above

        acc0 = tuple(jnp.zeros((_L,), jnp.float32) for _ in range(nw))
        acc = lax.fori_loop(0, _NQ1, quad_step, acc0)

        sc_cp.wait()
        plsc.parallel_loop(0, 256, 1, unroll=8)(zero_hist)

        def quad_hist(p, a):
            a = quad_step(p, a)
            lo = (p - q_hist0) * sl_h
            plsc.parallel_loop(lo, jnp.minimum(lo + sl_h, nvec), 1,
                               unroll=_NB)(h1)
            return a
        acc = lax.fori_loop(q_hist0, q_coll0, quad_hist, acc)

        t1, ca1 = scan_hist(jnp.int32(_TOPK))

        def coll(i, off):
            uu = keybits(i)
            skey = plsc.bitcast(uu, jnp.int32) ^ jnp.int32(_IMIN)
            bin1 = (uu >> 24).astype(jnp.int32)
            keep = bin1 >= t1
            plsc.store_compressed(cand_key_v.at[pl.ds(off, _L)], skey,
                                  mask=keep)
            plsc.store_compressed(cand_idx_v.at[pl.ds(off, _L)],
                                  lane + i * _L, mask=keep)
            cnt = plsc.all_reduce_population_count(keep)[0]
            return jnp.minimum(off + cnt, _CAND_CAP)

        def quad_coll(p, carry):
            a, off = carry
            a = quad_step(p, a)
            lo = (p - q_coll0) * sl_c
            off = plsc.parallel_loop(lo, jnp.minimum(lo + sl_c, nvec), 1,
                                     unroll=4, carry=off)(coll)
            return a, off
        acc, C = lax.fori_loop(q_coll0, nquad, quad_coll,
                               (acc, jnp.int32(0)))
        ctx = [a * (1.0 / T) for a in acc]

        cand_key_v[pl.ds(C, _L)] = jnp.full((_L,), _IMIN, jnp.int32)
        cand_idx_v[pl.ds(C, _L)] = jnp.full((_L,), _IBIG, jnp.int32)
        ncv = (C + _L - 1) // _L

        plsc.parallel_loop(0, 256, 1, unroll=8)(zero_hist)

        def lh(i):
            skey = cand_key_v[pl.ds(i * _L, _L)]
            uu = plsc.bitcast(skey ^ jnp.int32(_IMIN), jnp.uint32)
            bin1 = (uu >> 24).astype(jnp.int32)
            bin2 = ((uu >> 16).astype(jnp.int32)) & 255
            plsc.addupdate_scatter(hist_v, [bin2 * _L + lane], ones_i,
                                   mask=bin1 == t1)
        plsc.parallel_loop(0, ncv, 1, unroll=4)(lh)

        t2, _ca2 = scan_hist(jnp.int32(_TOPK) - ca1)
        thresh16 = t1 * 256 + t2

        def fc(i, off):
            skey = cand_key_v[pl.ds(i * _L, _L)]
            idx = cand_idx_v[pl.ds(i * _L, _L)]
            uu = plsc.bitcast(skey ^ jnp.int32(_IMIN), jnp.uint32)
            key16 = (uu >> 16).astype(jnp.int32)
            keep = key16 >= thresh16
            plsc.store_compressed(fkey_v.at[pl.ds(off, _L)], skey, mask=keep)
            plsc.store_compressed(fidx_v.at[pl.ds(off, _L)], idx, mask=keep)
            cnt = plsc.all_reduce_population_count(keep)[0]
            return jnp.minimum(off + cnt, _FCAP)
        C2 = plsc.parallel_loop(0, ncv, 1, unroll=4,
                                carry=jnp.int32(0))(fc)

        fkey_v[pl.ds(C2, _L)] = jnp.full((_L,), _IMIN, jnp.int32)
        fidx_v[pl.ds(C2, _L)] = jnp.full((_L,), _IBIG, jnp.int32)
        nv2 = (C2 + _L - 1) // _L

        p0 = lane * 32 + b
        topk_idx_v[pl.ds(0, _L)] = p0
        topk_idx_v[pl.ds(_L, _L)] = p0 + 512

        def pick(k, _):
            def scan_c(i, carry):
                bk, bi, bpos = carry
                v = fkey_v[pl.ds(i * _L, _L)]
                ix = fidx_v[pl.ds(i * _L, _L)]
                m = jnp.max(v)
                im = jnp.min(jnp.where(v == m, ix, jnp.int32(_IBIG)))
                pos = jnp.min(jnp.where(
                    jnp.logical_and(v == m, ix == im), lane + i * _L,
                    jnp.int32(_IBIG)))
                better = jnp.logical_or(
                    m > bk, jnp.logical_and(m == bk, im < bi))
                return (jnp.where(better, m, bk),
                        jnp.where(better, im, bi),
                        jnp.where(better, pos, bpos))
            bk, bi, bpos = plsc.parallel_loop(
                0, nv2, 1, unroll=2,
                carry=(jnp.int32(_IMIN), jnp.int32(_IBIG),
                       jnp.int32(_IBIG)))(scan_c)
            plsc.store_scatter(topk_idx_v, [jnp.broadcast_to(k, (_L,))],
                               jnp.broadcast_to(bi, (_L,)), mask=lane0)
            plsc.store_scatter(fkey_v, [jnp.broadcast_to(bpos, (_L,))],
                               jnp.full((_L,), _IMIN, jnp.int32), mask=lane0)
            return 0
        lax.fori_loop(0, _TOPK, pick, 0)

        pltpu.async_copy(emb_hbm.at[topk_idx_v], cand_rows_v, sem_g0).wait()
        wj = [w_v[pl.ds(j * _L, _L)] for j in range(nw)]

        def val_body(k, _):
            accv = jnp.zeros((_L,), jnp.float32)
            for j in range(nw):
                x = cand_rows_v[k, pl.ds(j * _L, _L)] + ctx[j]
                e = jnp.exp(x + x)
                th = 1.0 - 2.0 / (e + 1.0)
                accv = accv + th * wj[j]
            vk = jnp.sum(accv)
            plsc.store_scatter(values_v, [jnp.broadcast_to(k, (_L,))],
                               jnp.broadcast_to(vk, (_L,)), mask=lane0)
            return 0
        lax.fori_loop(0, _TOPK, val_body, 0)

        v0 = values_v[pl.ds(0, _L)]
        v1 = values_v[pl.ds(_L, _L)]
        mask4 = lane < (_TOPK - _L)
        tot = jnp.sum(v0) + jnp.sum(jnp.where(mask4, v1, 0.0))
        mean = tot * (1.0 / _TOPK)
        i0 = topk_idx_v[pl.ds(0, _L)]
        i1 = topk_idx_v[pl.ds(_L, _L)]
        plsc.addupdate_scatter(scores_v, [i0], (v0 - mean) * _BETA)
        plsc.addupdate_scatter(scores_v, [i1], (v1 - mean) * _BETA,
                               mask=mask4)

        pltpu.sync_copy(scores_v, out_hbm.at[b])

    return body


def kernel(input_ids, scores, emb, w):
    B, V = scores.shape
    T = input_ids.shape[1]
    D = emb.shape[1]
    NC, NS = 2, 16
    assert B == NC * NS
    mesh = plsc.VectorSubcoreMesh(core_axis_name="c", subcore_axis_name="s",
                                  num_cores=NC, num_subcores=NS)
    scratch = [
        pltpu.VMEM((T,), jnp.int32),
        pltpu.VMEM((V,), jnp.float32),
        pltpu.VMEM((_CH, D), jnp.float32),
        pltpu.VMEM((_CH, D), jnp.float32),
        pltpu.VMEM((_CH, D), jnp.float32),
        pltpu.VMEM((_CH, D), jnp.float32),
        pltpu.VMEM((2 * _L, D), jnp.float32),
        pltpu.VMEM((256 * _L,), jnp.int32),
        pltpu.VMEM((_CAND_CAP + _L,), jnp.int32),
        pltpu.VMEM((_CAND_CAP + _L,), jnp.int32),
        pltpu.VMEM((_FCAP + _L,), jnp.int32),
        pltpu.VMEM((_FCAP + _L,), jnp.int32),
        pltpu.VMEM((2 * _L,), jnp.int32),
        pltpu.VMEM((2 * _L,), jnp.float32),
        pltpu.VMEM((D,), jnp.float32),
        pltpu.SemaphoreType.DMA,
        pltpu.SemaphoreType.DMA,
        pltpu.SemaphoreType.DMA,
        pltpu.SemaphoreType.DMA,
        pltpu.SemaphoreType.DMA,
    ]
    run = pl.kernel(_sc_body(V, T, D, NC, NS),
                    out_type=jax.ShapeDtypeStruct((B, V), jnp.float32),
                    mesh=mesh, scratch_types=scratch,
                    compiler_params=pltpu.CompilerParams(
                        needs_layout_passes=False))
    return run(input_ids.astype(jnp.int32), scores, emb, w)
